# Initial kernel scaffold; baseline (speedup 1.0000x reference)
import jax
import jax.numpy as jnp
from jax.experimental import pallas as pl


def kernel(x, edge_index, batch, W1, b1, W2, b2, W3, b3, lin1_W, lin1_b, lin2_W, lin2_b, lin3_W, lin3_b):
    raise NotImplementedError("write your pallas kernel here")


# speedup vs baseline: 1.6347x; 1.6347x over previous
import functools
import math

import jax
import jax.numpy as jnp
from jax import lax
from jax.experimental import pallas as pl
from jax.experimental.pallas import tpu as pltpu
from jax.experimental.pallas import tpu_sc as plsc

N_NODES = 10000
N_EDGES = 320000
D = 128
NW = 32
EC = 128
E_PAD = 327680
NBLK = E_PAD // EC
PER_W = NBLK // NW

_mesh = functools.partial(
    plsc.VectorSubcoreMesh, core_axis_name="c", subcore_axis_name="s")


def _pad_rows(n):
    return ((n + 255) // 256) * 256


@functools.lru_cache(None)
def _seg_count(n_pad):
    zr = n_pad // 16

    @functools.partial(
        pl.kernel, mesh=_mesh(),
        out_type=jax.ShapeDtypeStruct((2 * n_pad,), jnp.float32),
        scratch_types=[
            pltpu.VMEM((EC,), jnp.int32),
            pltpu.VMEM((EC,), jnp.float32),
            pltpu.VMEM((n_pad // 16,), jnp.float32),
            pltpu.VMEM_SHARED((n_pad,), jnp.float32),
        ],
    )
    def k(idx_hbm, ones_hbm, zeros_hbm, out, idx_v, ones_v, buf_v, acc):
        c = lax.axis_index("c")
        s = lax.axis_index("s")
        w = c * 16 + s
        pltpu.sync_copy(zeros_hbm.at[pl.ds(s * zr, zr)], buf_v)
        pltpu.sync_copy(buf_v, acc.at[pl.ds(s * zr, zr)])
        pltpu.sync_copy(ones_hbm, ones_v)
        plsc.subcore_barrier()

        def body(j, carry):
            pltpu.sync_copy(idx_hbm.at[pl.ds((w * PER_W + j) * EC, EC)], idx_v)
            pltpu.sync_copy(ones_v, acc.at[idx_v], add=True)
            return carry

        lax.fori_loop(0, PER_W, body, 0)
        plsc.subcore_barrier()
        pltpu.sync_copy(acc.at[pl.ds(s * zr, zr)], buf_v)
        pltpu.sync_copy(buf_v, out.at[pl.ds(c * n_pad + s * zr, zr)])

    return k


@functools.lru_cache(None)
def _row_agg(n_tab, n_out_pad):
    zr = n_out_pad // 16

    @functools.partial(
        pl.kernel, mesh=_mesh(),
        out_type=jax.ShapeDtypeStruct((2, n_out_pad, D), jnp.float32),
        scratch_types=[
            pltpu.VMEM((EC,), jnp.int32),
            pltpu.VMEM((EC,), jnp.int32),
            pltpu.VMEM((EC, D), jnp.float32),
            pltpu.VMEM_SHARED((n_out_pad, D), jnp.float32),
            pltpu.SemaphoreType.DMA,
        ],
    )
    def k(tab_hbm, src_hbm, dst_hbm, zeros_hbm, out,
          src_v, dst_v, rows_v, acc, sem):
        c = lax.axis_index("c")
        s = lax.axis_index("s")
        w = c * 16 + s
        pltpu.sync_copy(zeros_hbm.at[pl.ds(s * zr, zr)], acc.at[pl.ds(s * zr, zr)])
        plsc.subcore_barrier()

        def body(j, carry):
            base = (w * PER_W + j) * EC
            pltpu.sync_copy(src_hbm.at[pl.ds(base, EC)], src_v)
            pltpu.sync_copy(dst_hbm.at[pl.ds(base, EC)], dst_v)
            pltpu.async_copy(tab_hbm.at[src_v], rows_v, sem).wait()
            pltpu.sync_copy(rows_v, acc.at[dst_v], add=True)
            return carry

        lax.fori_loop(0, PER_W, body, 0)
        plsc.subcore_barrier()
        pltpu.sync_copy(acc.at[pl.ds(s * zr, zr)], out.at[c, pl.ds(s * zr, zr)])

    return k


@functools.lru_cache(None)
def _row_compact(n_pad, k_pad):
    T = 5
    ch = n_pad // (NW * T)
    zr = k_pad // 16

    @functools.partial(
        pl.kernel, mesh=_mesh(),
        out_type=jax.ShapeDtypeStruct((2, k_pad, D), jnp.float32),
        scratch_types=[
            pltpu.VMEM((ch,), jnp.int32),
            pltpu.VMEM((ch, D), jnp.float32),
            pltpu.VMEM_SHARED((k_pad, D), jnp.float32),
        ],
    )
    def k(h_hbm, posx_hbm, zeros_hbm, out, posx_v, rows_v, acc):
        c = lax.axis_index("c")
        s = lax.axis_index("s")
        w = c * 16 + s
        pltpu.sync_copy(zeros_hbm.at[pl.ds(s * zr, zr)], acc.at[pl.ds(s * zr, zr)])
        plsc.subcore_barrier()

        def body(t, carry):
            base = w * (T * ch) + t * ch
            pltpu.sync_copy(posx_hbm.at[pl.ds(base, ch)], posx_v)
            pltpu.sync_copy(h_hbm.at[pl.ds(base, ch)], rows_v)
            pltpu.sync_copy(rows_v, acc.at[posx_v], add=True)
            return carry

        lax.fori_loop(0, T, body, 0)
        plsc.subcore_barrier()
        pltpu.sync_copy(acc.at[pl.ds(s * zr, zr)], out.at[c, pl.ds(s * zr, zr)])

    return k


@functools.lru_cache(None)
def _elem_gather(n_tab):
    @functools.partial(
        pl.kernel, mesh=_mesh(),
        out_type=jax.ShapeDtypeStruct((E_PAD,), jnp.int32),
        scratch_types=[
            pltpu.VMEM((EC,), jnp.int32),
            pltpu.VMEM((EC,), jnp.int32),
            pltpu.SemaphoreType.DMA,
        ],
    )
    def k(tab_hbm, idx_hbm, out, idx_v, g_v, sem):
        c = lax.axis_index("c")
        s = lax.axis_index("s")
        w = c * 16 + s

        def body(j, carry):
            base = (w * PER_W + j) * EC
            pltpu.sync_copy(idx_hbm.at[pl.ds(base, EC)], idx_v)
            pltpu.async_copy(tab_hbm.at[idx_v], g_v, sem).wait()
            pltpu.sync_copy(g_v, out.at[pl.ds(base, EC)])
            return carry

        lax.fori_loop(0, PER_W, body, 0)

    return k


def _tc(body, out_shape):
    return pl.pallas_call(body, out_shape=out_shape)


def _mm(x, w):
    def body(x_ref, w_ref, o_ref):
        o_ref[...] = jnp.dot(x_ref[...], w_ref[...],
                             preferred_element_type=jnp.float32)
    return _tc(body, jax.ShapeDtypeStruct((x.shape[0], w.shape[1]),
                                          jnp.float32))(x, w)


def _pre(deg2, h):
    n_pad = h.shape[0]

    def body(d_ref, h_ref, g_ref, di_ref, di2_ref):
        d2 = d_ref[0, :] + d_ref[1, :]
        dinv = lax.rsqrt(d2 + 1.0)
        dinv2 = jnp.where(d2 > 0, lax.rsqrt(jnp.where(d2 > 0, d2, 1.0)), 0.0)
        di_ref[...] = dinv
        di2_ref[...] = dinv2
        g_ref[...] = dinv[:, None] * h_ref[...]

    return _tc(body, (jax.ShapeDtypeStruct((n_pad, D), jnp.float32),
                      jax.ShapeDtypeStruct((n_pad,), jnp.float32),
                      jax.ShapeDtypeStruct((n_pad,), jnp.float32)))(deg2, h)


def _post_gcn(A, g, dinv, b, n):
    n_pad = g.shape[0]

    def body(a_ref, g_ref, di_ref, b_ref, o_ref):
        acc = a_ref[0] + a_ref[1] + g_ref[...]
        h = jax.nn.relu(di_ref[...][:, None] * acc + b_ref[...][None, :])
        row = lax.broadcasted_iota(jnp.int32, (n_pad, D), 0)
        o_ref[...] = jnp.where(row < n, h, 0.0)

    return _tc(body, jax.ShapeDtypeStruct((n_pad, D), jnp.float32))(
        A, g, dinv, b)


def _g2(h, dinv2):
    n_pad = h.shape[0]

    def body(h_ref, di2_ref, o_ref):
        o_ref[...] = di2_ref[...][:, None] * h_ref[...]

    return _tc(body, jax.ShapeDtypeStruct((n_pad, D), jnp.float32))(h, dinv2)


def _score(h, B, dinv2, n):
    n_pad = h.shape[0]
    R = n_pad // 128

    def body(h_ref, b_ref, di2_ref, o_ref):
        agg = h_ref[...] - di2_ref[...][:, None] * (b_ref[0] + b_ref[1])
        sc = jnp.sum(jnp.abs(agg), axis=1).reshape(R, 128)
        idx = (lax.broadcasted_iota(jnp.int32, (R, 128), 0) * 128
               + lax.broadcasted_iota(jnp.int32, (R, 128), 1))
        o_ref[...] = jnp.where(idx < n, sc, -jnp.inf)

    return _tc(body, jax.ShapeDtypeStruct((R, 128), jnp.float32))(h, B, dinv2)


def _prefix_excl(m):
    R = m.shape[0]
    jj = lax.broadcasted_iota(jnp.int32, (128, 128), 0)
    kk = lax.broadcasted_iota(jnp.int32, (128, 128), 1)
    lincl = (jj <= kk).astype(jnp.float32)
    lane_incl = jnp.dot(m, lincl, preferred_element_type=jnp.float32)
    lane_excl = lane_incl - m
    totals = jnp.sum(m, axis=1)
    totv = jnp.concatenate([totals, jnp.zeros((128 - R,), jnp.float32)])
    strict = (kk < jj).astype(jnp.float32)
    offs = jnp.sum(strict * totv[None, :], axis=1)[:R]
    return offs[:, None] + lane_excl


def _select(score, n, k):
    R = score.shape[0]

    def body(s_ref, posx_ref, mval_ref):
        s = s_ref[...]
        idx = (lax.broadcasted_iota(jnp.int32, (R, 128), 0) * 128
               + lax.broadcasted_iota(jnp.int32, (R, 128), 1))
        mask = idx < n
        key = lax.bitcast_convert_type(s, jnp.int32)
        key = jnp.where(mask, key, -1)

        def step(_, lohi):
            lo, hi = lohi
            d = hi - lo
            mid = lo + d // 2 + (d & 1)
            cnt = jnp.sum((key >= mid).astype(jnp.float32))
            big = cnt >= k
            return (jnp.where(big, mid, lo), jnp.where(big, hi, mid - 1))

        lo, hi = lax.fori_loop(0, 31, step, (jnp.int32(0), jnp.int32(2**31 - 1)))
        tau = lo
        gt = key > tau
        cnt_gt = jnp.sum(gt.astype(jnp.float32))
        tie = key == tau
        tie_rank = _prefix_excl(tie.astype(jnp.float32))
        selected = gt | (tie & (tie_rank < (k - cnt_gt)))
        mapping = _prefix_excl(selected.astype(jnp.float32)).astype(jnp.int32)
        posx_ref[...] = jnp.where(selected, mapping, k)
        mval_ref[...] = jnp.where(selected, mapping, -1)

    return _tc(body, (jax.ShapeDtypeStruct((R, 128), jnp.int32),
                      jax.ShapeDtypeStruct((R, 128), jnp.int32)))(score)


def _relabel(a, b, ew, zrow):
    E2 = a.shape

    def body(a_ref, b_ref, w_ref, s_ref, d_ref, wo_ref, sx_ref):
        av, bv, wv = a_ref[...], b_ref[...], w_ref[...]
        valid = (av >= 0) & (bv >= 0) & (wv > 0)
        s_ref[...] = jnp.where(valid, av, 0)
        d_ref[...] = jnp.where(valid, bv, 0)
        wo_ref[...] = valid.astype(jnp.float32)
        sx_ref[...] = jnp.where(valid, av, zrow)

    return _tc(body, (jax.ShapeDtypeStruct(E2, jnp.int32),
                      jax.ShapeDtypeStruct(E2, jnp.int32),
                      jax.ShapeDtypeStruct(E2, jnp.float32),
                      jax.ShapeDtypeStruct(E2, jnp.int32)))(a, b, ew)


def _readout_combine(Xh, kk):
    n_pad = Xh.shape[1]

    def body(x_ref, ro_ref, xo_ref):
        x = x_ref[0] + x_ref[1]
        row = lax.broadcasted_iota(jnp.int32, (n_pad, D), 0)
        xm = jnp.where(row < kk, x, 0.0)
        xo_ref[...] = xm
        mx = jnp.max(jnp.where(row < kk, x, -jnp.inf), axis=0)
        mn = jnp.sum(xm, axis=0) / kk
        ro_ref[...] = jnp.concatenate([mx, mn]).reshape(1, 2 * D)

    return _tc(body, (jax.ShapeDtypeStruct((1, 2 * D), jnp.float32),
                      jax.ShapeDtypeStruct((n_pad, D), jnp.float32)))(Xh)


def _readout_single(X, kk):
    n_pad = X.shape[0]

    def body(x_ref, ro_ref):
        x = x_ref[...]
        row = lax.broadcasted_iota(jnp.int32, (n_pad, D), 0)
        mx = jnp.max(jnp.where(row < kk, x, -jnp.inf), axis=0)
        mn = jnp.sum(jnp.where(row < kk, x, 0.0), axis=0) / kk
        ro_ref[...] = jnp.concatenate([mx, mn]).reshape(1, 2 * D)

    return _tc(body, jax.ShapeDtypeStruct((1, 2 * D), jnp.float32))(X)


def _head(x1, x2, x3, lin1_W, lin1_b, lin2_W, lin2_b, lin3_W, lin3_b):
    def body(x1_ref, x2_ref, x3_ref, w1_ref, b1_ref, w2_ref, b2_ref,
             w3_ref, b3_ref, o_ref):
        z = (jax.nn.relu(x1_ref[...]) + jax.nn.relu(x2_ref[...])
             + jax.nn.relu(x3_ref[...]))
        z = jax.nn.relu(jnp.dot(z, w1_ref[...],
                                preferred_element_type=jnp.float32)
                        + b1_ref[...][None, :])
        z = jax.nn.relu(jnp.dot(z, w2_ref[...],
                                preferred_element_type=jnp.float32)
                        + b2_ref[...][None, :])
        z = (jnp.dot(z, w3_ref[...], preferred_element_type=jnp.float32)
             + b3_ref[...][None, :])
        m = jnp.max(z, axis=-1, keepdims=True)
        e = jnp.exp(z - m)
        o_ref[...] = z - m - jnp.log(jnp.sum(e, axis=-1, keepdims=True))

    return _tc(body, jax.ShapeDtypeStruct((1, 6), jnp.float32))(
        x1, x2, x3, lin1_W, lin1_b, lin2_W, lin2_b, lin3_W, lin3_b)


ER = E_PAD // 128


def _layer(x_pad, W, b, srcx, dst, n):
    n_pad = x_pad.shape[0]
    zeros1 = jnp.zeros((n_pad,), jnp.float32)
    zeros2 = jnp.zeros((n_pad, D), jnp.float32)
    ones_ec = jnp.ones((EC,), jnp.float32)
    h = _mm(x_pad, W)
    deg = _seg_count(n_pad)(srcx, ones_ec, zeros1).reshape(2, n_pad)
    g, dinv, dinv2 = _pre(deg, h)
    A = _row_agg(n_pad, n_pad)(g, srcx, dst, zeros2)
    h_out = _post_gcn(A, g, dinv, b, n)
    return h_out, dinv2


def _pool(h_out, dinv2, srcx, dst, src_flat, dst_flat, ew_flat, n, kk):
    n_pad = h_out.shape[0]
    k_pad = _pad_rows(kk)
    g2 = _g2(h_out, dinv2)
    B = _row_agg(n_pad, n_pad)(g2, srcx, dst,
                               jnp.zeros((n_pad, D), jnp.float32))
    sc = _score(h_out, B, dinv2, n)
    posx, mval = _select(sc, n, kk)
    Xh = _row_compact(n_pad, k_pad)(h_out, posx.reshape(n_pad),
                                    jnp.zeros((k_pad, D), jnp.float32))
    ro, X = _readout_combine(Xh, kk)
    mval_flat = mval.reshape(n_pad)
    a = _elem_gather(n_pad)(mval_flat, src_flat)
    bg = _elem_gather(n_pad)(mval_flat, dst_flat)
    s_n, d_n, ew_n, sx_n = _relabel(a.reshape(ER, 128),
                                    bg.reshape(ER, 128),
                                    ew_flat.reshape(ER, 128), kk)
    return (ro, X, s_n.reshape(-1), d_n.reshape(-1), ew_n.reshape(-1),
            sx_n.reshape(-1))


def kernel(x, edge_index, batch, W1, b1, W2, b2, W3, b3,
           lin1_W, lin1_b, lin2_W, lin2_b, lin3_W, lin3_b):
    n = N_NODES
    n_pad = _pad_rows(n)
    pad_e = E_PAD - N_EDGES
    src = jnp.pad(edge_index[0].astype(jnp.int32), (0, pad_e),
                  constant_values=n)
    dst = jnp.pad(edge_index[1].astype(jnp.int32), (0, pad_e))
    ew = jnp.pad(jnp.ones((N_EDGES,), jnp.float32), (0, pad_e))
    x_pad = jnp.pad(x, ((0, n_pad - n), (0, 0)))

    h1, dinv21 = _layer(x_pad, W1, b1, src, dst, n)
    x1ro, X1, s1, d1, ew1, sx1 = _pool(
        h1, dinv21, src, dst, src, dst, ew, n, 5000)

    k1 = 5000
    h2, dinv22 = _layer(X1, W2, b2, sx1, d1, k1)
    x2ro, X2, s2, d2, ew2, sx2 = _pool(
        h2, dinv22, sx1, d1, s1, d1, ew1, k1, 2500)

    k2 = 2500
    h3, _ = _layer(X2, W3, b3, sx2, d2, k2)
    x3ro = _readout_single(h3, k2)

    return _head(x1ro, x2ro, x3ro, lin1_W, lin1_b, lin2_W, lin2_b,
                 lin3_W, lin3_b)


# speedup vs baseline: 1.6440x; 1.0057x over previous
import functools
import math

import jax
import jax.numpy as jnp
from jax import lax
from jax.experimental import pallas as pl
from jax.experimental.pallas import tpu as pltpu
from jax.experimental.pallas import tpu_sc as plsc

N_NODES = 10000
N_EDGES = 320000
D = 128
NW = 32
EC = 128
E_PAD = 327680
NBLK = E_PAD // EC
PER_W = NBLK // NW

_mesh = functools.partial(
    plsc.VectorSubcoreMesh, core_axis_name="c", subcore_axis_name="s")


def _pad_rows(n):
    return ((n + 255) // 256) * 256


@functools.lru_cache(None)
def _seg_count(n_pad):
    zr = n_pad // 16
    NB = 8
    G = PER_W // NB

    @functools.partial(
        pl.kernel, mesh=_mesh(),
        out_type=jax.ShapeDtypeStruct((2 * n_pad,), jnp.float32),
        scratch_types=(
            [pltpu.VMEM((EC,), jnp.int32) for _ in range(NB)]
            + [pltpu.VMEM((EC,), jnp.float32),
               pltpu.VMEM((n_pad // 16,), jnp.float32),
               pltpu.VMEM_SHARED((n_pad,), jnp.float32),
               pltpu.SemaphoreType.DMA,
               pltpu.SemaphoreType.DMA]
        ),
    )
    def k(idx_hbm, ones_hbm, zeros_hbm, out, *rest):
        idx_vs = rest[:NB]
        ones_v, buf_v, acc, sem_i, sem_s = rest[NB:]
        c = lax.axis_index("c")
        s = lax.axis_index("s")
        w = c * 16 + s
        pltpu.sync_copy(zeros_hbm.at[pl.ds(s * zr, zr)], buf_v)
        pltpu.sync_copy(buf_v, acc.at[pl.ds(s * zr, zr)])
        pltpu.sync_copy(ones_hbm, ones_v)
        plsc.subcore_barrier()

        def body(g, carry):
            j0 = (w * PER_W + g * NB) * EC
            loads = [pltpu.async_copy(
                idx_hbm.at[pl.ds(j0 + b * EC, EC)], idx_vs[b], sem_i)
                for b in range(NB)]
            for cp in loads:
                cp.wait()
            for b in range(NB):
                pltpu.sync_copy(ones_v, acc.at[idx_vs[b]], add=True)
            return carry

        lax.fori_loop(0, G, body, 0)
        plsc.subcore_barrier()
        pltpu.sync_copy(acc.at[pl.ds(s * zr, zr)], buf_v)
        pltpu.sync_copy(buf_v, out.at[pl.ds(c * n_pad + s * zr, zr)])

    return k


@functools.lru_cache(None)
def _row_agg(n_tab, n_out_pad):
    zr = n_out_pad // 16
    ECR = 64
    PER = E_PAD // (NW * ECR)
    NB = 5
    G = PER // NB

    @functools.partial(
        pl.kernel, mesh=_mesh(),
        out_type=jax.ShapeDtypeStruct((2, n_out_pad, D), jnp.float32),
        scratch_types=(
            [pltpu.VMEM((ECR,), jnp.int32) for _ in range(2 * NB)]
            + [pltpu.VMEM((ECR, D), jnp.float32) for _ in range(NB)]
            + [pltpu.VMEM_SHARED((n_out_pad, D), jnp.float32),
               pltpu.SemaphoreType.DMA,
               pltpu.SemaphoreType.DMA]
        ),
    )
    def k(tab_hbm, src_hbm, dst_hbm, zeros_hbm, out, *rest):
        src_vs = rest[:NB]
        dst_vs = rest[NB:2 * NB]
        rows_vs = rest[2 * NB:3 * NB]
        acc, sem_i, sem_g = rest[3 * NB:]
        c = lax.axis_index("c")
        s = lax.axis_index("s")
        w = c * 16 + s
        pltpu.sync_copy(zeros_hbm.at[pl.ds(s * zr, zr)], acc.at[pl.ds(s * zr, zr)])
        plsc.subcore_barrier()

        def body(g, carry):
            j0 = (w * PER + g * NB) * ECR
            loads = [pltpu.async_copy(
                src_hbm.at[pl.ds(j0 + b * ECR, ECR)], src_vs[b], sem_i)
                for b in range(NB)]
            loads += [pltpu.async_copy(
                dst_hbm.at[pl.ds(j0 + b * ECR, ECR)], dst_vs[b], sem_i)
                for b in range(NB)]
            for cp in loads:
                cp.wait()
            gathers = [pltpu.async_copy(tab_hbm.at[src_vs[b]], rows_vs[b],
                                        sem_g) for b in range(NB)]
            for cp in gathers:
                cp.wait()
            for b in range(NB):
                pltpu.sync_copy(rows_vs[b], acc.at[dst_vs[b]], add=True)
            return carry

        lax.fori_loop(0, G, body, 0)
        plsc.subcore_barrier()
        pltpu.sync_copy(acc.at[pl.ds(s * zr, zr)], out.at[c, pl.ds(s * zr, zr)])

    return k


@functools.lru_cache(None)
def _row_compact(n_pad, k_pad):
    T = 5
    ch = n_pad // (NW * T)
    zr = k_pad // 16

    @functools.partial(
        pl.kernel, mesh=_mesh(),
        out_type=jax.ShapeDtypeStruct((2, k_pad, D), jnp.float32),
        scratch_types=(
            [pltpu.VMEM((ch,), jnp.int32) for _ in range(T)]
            + [pltpu.VMEM((ch, D), jnp.float32) for _ in range(T)]
            + [pltpu.VMEM_SHARED((k_pad, D), jnp.float32),
               pltpu.SemaphoreType.DMA,
               pltpu.SemaphoreType.DMA]
        ),
    )
    def k(h_hbm, posx_hbm, zeros_hbm, out, *rest):
        posx_vs = rest[:T]
        rows_vs = rest[T:2 * T]
        acc, sem_i, sem_s = rest[2 * T:]
        c = lax.axis_index("c")
        s = lax.axis_index("s")
        w = c * 16 + s
        pltpu.sync_copy(zeros_hbm.at[pl.ds(s * zr, zr)], acc.at[pl.ds(s * zr, zr)])
        plsc.subcore_barrier()
        loads = [pltpu.async_copy(
            posx_hbm.at[pl.ds(w * (T * ch) + t * ch, ch)], posx_vs[t], sem_i)
            for t in range(T)]
        loads += [pltpu.async_copy(
            h_hbm.at[pl.ds(w * (T * ch) + t * ch, ch)], rows_vs[t], sem_i)
            for t in range(T)]
        for cp in loads:
            cp.wait()
        for t in range(T):
            pltpu.sync_copy(rows_vs[t], acc.at[posx_vs[t]], add=True)
        plsc.subcore_barrier()
        pltpu.sync_copy(acc.at[pl.ds(s * zr, zr)], out.at[c, pl.ds(s * zr, zr)])

    return k


@functools.lru_cache(None)
def _elem_gather2(n_tab):
    NB = 8
    G = (2 * PER_W) // NB

    @functools.partial(
        pl.kernel, mesh=_mesh(),
        out_type=jax.ShapeDtypeStruct((2 * E_PAD,), jnp.int32),
        scratch_types=(
            [pltpu.VMEM((EC,), jnp.int32) for _ in range(2 * NB)]
            + [pltpu.SemaphoreType.DMA,
               pltpu.SemaphoreType.DMA,
               pltpu.SemaphoreType.DMA]
        ),
    )
    def k(tab_hbm, src_hbm, dst_hbm, out, *rest):
        idx_vs = rest[:NB]
        g_vs = rest[NB:2 * NB]
        sem_i, sem_g, sem_o = rest[2 * NB:]
        c = lax.axis_index("c")
        s = lax.axis_index("s")
        w = c * 16 + s

        def make_body(idx_ref, half):
            def body(g, carry):
                base0 = (w * PER_W + g * NB) * EC
                loads = [pltpu.async_copy(
                    idx_ref.at[pl.ds(base0 + b * EC, EC)], idx_vs[b], sem_i)
                    for b in range(NB)]
                for cp in loads:
                    cp.wait()
                gathers = [pltpu.async_copy(tab_hbm.at[idx_vs[b]], g_vs[b],
                                            sem_g) for b in range(NB)]
                for cp in gathers:
                    cp.wait()
                stores = [pltpu.async_copy(
                    g_vs[b], out.at[pl.ds(half * E_PAD + base0 + b * EC, EC)],
                    sem_o) for b in range(NB)]
                for cp in stores:
                    cp.wait()
                return carry
            return body

        lax.fori_loop(0, PER_W // NB, make_body(src_hbm, 0), 0)
        lax.fori_loop(0, PER_W // NB, make_body(dst_hbm, 1), 0)

    return k


def _tc(body, out_shape):
    return pl.pallas_call(body, out_shape=out_shape)


def _mm(x, w):
    def body(x_ref, w_ref, o_ref):
        o_ref[...] = jnp.dot(x_ref[...], w_ref[...],
                             preferred_element_type=jnp.float32)
    return _tc(body, jax.ShapeDtypeStruct((x.shape[0], w.shape[1]),
                                          jnp.float32))(x, w)


def _pre(deg2, h):
    n_pad = h.shape[0]

    def body(d_ref, h_ref, g_ref, di_ref, di2_ref):
        d2 = d_ref[0, :] + d_ref[1, :]
        dinv = lax.rsqrt(d2 + 1.0)
        dinv2 = jnp.where(d2 > 0, lax.rsqrt(jnp.where(d2 > 0, d2, 1.0)), 0.0)
        di_ref[...] = dinv
        di2_ref[...] = dinv2
        g_ref[...] = dinv[:, None] * h_ref[...]

    return _tc(body, (jax.ShapeDtypeStruct((n_pad, D), jnp.float32),
                      jax.ShapeDtypeStruct((n_pad,), jnp.float32),
                      jax.ShapeDtypeStruct((n_pad,), jnp.float32)))(deg2, h)


def _post_gcn(A, g, dinv, b, n):
    n_pad = g.shape[0]

    def body(a_ref, g_ref, di_ref, b_ref, o_ref):
        acc = a_ref[0] + a_ref[1] + g_ref[...]
        h = jax.nn.relu(di_ref[...][:, None] * acc + b_ref[...][None, :])
        row = lax.broadcasted_iota(jnp.int32, (n_pad, D), 0)
        o_ref[...] = jnp.where(row < n, h, 0.0)

    return _tc(body, jax.ShapeDtypeStruct((n_pad, D), jnp.float32))(
        A, g, dinv, b)


def _g2(h, dinv2):
    n_pad = h.shape[0]

    def body(h_ref, di2_ref, o_ref):
        o_ref[...] = di2_ref[...][:, None] * h_ref[...]

    return _tc(body, jax.ShapeDtypeStruct((n_pad, D), jnp.float32))(h, dinv2)


def _score(h, B, dinv2, n):
    n_pad = h.shape[0]
    R = n_pad // 128

    def body(h_ref, b_ref, di2_ref, o_ref):
        agg = h_ref[...] - di2_ref[...][:, None] * (b_ref[0] + b_ref[1])
        sc = jnp.sum(jnp.abs(agg), axis=1).reshape(R, 128)
        idx = (lax.broadcasted_iota(jnp.int32, (R, 128), 0) * 128
               + lax.broadcasted_iota(jnp.int32, (R, 128), 1))
        o_ref[...] = jnp.where(idx < n, sc, -jnp.inf)

    return _tc(body, jax.ShapeDtypeStruct((R, 128), jnp.float32))(h, B, dinv2)


def _prefix_excl(m):
    R = m.shape[0]
    jj = lax.broadcasted_iota(jnp.int32, (128, 128), 0)
    kk = lax.broadcasted_iota(jnp.int32, (128, 128), 1)
    lincl = (jj <= kk).astype(jnp.float32)
    lane_incl = jnp.dot(m, lincl, preferred_element_type=jnp.float32)
    lane_excl = lane_incl - m
    totals = jnp.sum(m, axis=1)
    totv = jnp.concatenate([totals, jnp.zeros((128 - R,), jnp.float32)])
    strict = (kk < jj).astype(jnp.float32)
    offs = jnp.sum(strict * totv[None, :], axis=1)[:R]
    return offs[:, None] + lane_excl


def _select(score, n, k):
    R = score.shape[0]

    def body(s_ref, posx_ref, mval_ref):
        s = s_ref[...]
        idx = (lax.broadcasted_iota(jnp.int32, (R, 128), 0) * 128
               + lax.broadcasted_iota(jnp.int32, (R, 128), 1))
        mask = idx < n
        key = lax.bitcast_convert_type(s, jnp.int32)
        key = jnp.where(mask, key, -1)

        def step(_, lohi):
            lo, hi = lohi
            d = hi - lo
            mid = lo + d // 2 + (d & 1)
            cnt = jnp.sum((key >= mid).astype(jnp.float32))
            big = cnt >= k
            return (jnp.where(big, mid, lo), jnp.where(big, hi, mid - 1))

        lo, hi = lax.fori_loop(0, 31, step, (jnp.int32(0), jnp.int32(2**31 - 1)))
        tau = lo
        gt = key > tau
        cnt_gt = jnp.sum(gt.astype(jnp.float32))
        tie = key == tau
        tie_rank = _prefix_excl(tie.astype(jnp.float32))
        selected = gt | (tie & (tie_rank < (k - cnt_gt)))
        mapping = _prefix_excl(selected.astype(jnp.float32)).astype(jnp.int32)
        posx_ref[...] = jnp.where(selected, mapping, k)
        mval_ref[...] = jnp.where(selected, mapping, -1)

    return _tc(body, (jax.ShapeDtypeStruct((R, 128), jnp.int32),
                      jax.ShapeDtypeStruct((R, 128), jnp.int32)))(score)


def _relabel(a, b, ew, zrow):
    E2 = a.shape

    def body(a_ref, b_ref, w_ref, s_ref, d_ref, wo_ref, sx_ref):
        av, bv, wv = a_ref[...], b_ref[...], w_ref[...]
        valid = (av >= 0) & (bv >= 0) & (wv > 0)
        s_ref[...] = jnp.where(valid, av, 0)
        d_ref[...] = jnp.where(valid, bv, 0)
        wo_ref[...] = valid.astype(jnp.float32)
        sx_ref[...] = jnp.where(valid, av, zrow)

    return _tc(body, (jax.ShapeDtypeStruct(E2, jnp.int32),
                      jax.ShapeDtypeStruct(E2, jnp.int32),
                      jax.ShapeDtypeStruct(E2, jnp.float32),
                      jax.ShapeDtypeStruct(E2, jnp.int32)))(a, b, ew)


def _readout_combine(Xh, kk):
    n_pad = Xh.shape[1]

    def body(x_ref, ro_ref, xo_ref):
        x = x_ref[0] + x_ref[1]
        row = lax.broadcasted_iota(jnp.int32, (n_pad, D), 0)
        xm = jnp.where(row < kk, x, 0.0)
        xo_ref[...] = xm
        mx = jnp.max(jnp.where(row < kk, x, -jnp.inf), axis=0)
        mn = jnp.sum(xm, axis=0) / kk
        ro_ref[...] = jnp.concatenate([mx, mn]).reshape(1, 2 * D)

    return _tc(body, (jax.ShapeDtypeStruct((1, 2 * D), jnp.float32),
                      jax.ShapeDtypeStruct((n_pad, D), jnp.float32)))(Xh)


def _readout_single(X, kk):
    n_pad = X.shape[0]

    def body(x_ref, ro_ref):
        x = x_ref[...]
        row = lax.broadcasted_iota(jnp.int32, (n_pad, D), 0)
        mx = jnp.max(jnp.where(row < kk, x, -jnp.inf), axis=0)
        mn = jnp.sum(jnp.where(row < kk, x, 0.0), axis=0) / kk
        ro_ref[...] = jnp.concatenate([mx, mn]).reshape(1, 2 * D)

    return _tc(body, jax.ShapeDtypeStruct((1, 2 * D), jnp.float32))(X)


def _head(x1, x2, x3, lin1_W, lin1_b, lin2_W, lin2_b, lin3_W, lin3_b):
    def body(x1_ref, x2_ref, x3_ref, w1_ref, b1_ref, w2_ref, b2_ref,
             w3_ref, b3_ref, o_ref):
        z = (jax.nn.relu(x1_ref[...]) + jax.nn.relu(x2_ref[...])
             + jax.nn.relu(x3_ref[...]))
        z = jax.nn.relu(jnp.dot(z, w1_ref[...],
                                preferred_element_type=jnp.float32)
                        + b1_ref[...][None, :])
        z = jax.nn.relu(jnp.dot(z, w2_ref[...],
                                preferred_element_type=jnp.float32)
                        + b2_ref[...][None, :])
        z = (jnp.dot(z, w3_ref[...], preferred_element_type=jnp.float32)
             + b3_ref[...][None, :])
        m = jnp.max(z, axis=-1, keepdims=True)
        e = jnp.exp(z - m)
        o_ref[...] = z - m - jnp.log(jnp.sum(e, axis=-1, keepdims=True))

    return _tc(body, jax.ShapeDtypeStruct((1, 6), jnp.float32))(
        x1, x2, x3, lin1_W, lin1_b, lin2_W, lin2_b, lin3_W, lin3_b)


ER = E_PAD // 128


def _layer(x_pad, W, b, srcx, dst, n):
    n_pad = x_pad.shape[0]
    zeros1 = jnp.zeros((n_pad,), jnp.float32)
    zeros2 = jnp.zeros((n_pad, D), jnp.float32)
    ones_ec = jnp.ones((EC,), jnp.float32)
    h = _mm(x_pad, W)
    deg = _seg_count(n_pad)(srcx, ones_ec, zeros1).reshape(2, n_pad)
    g, dinv, dinv2 = _pre(deg, h)
    A = _row_agg(n_pad, n_pad)(g, srcx, dst, zeros2)
    h_out = _post_gcn(A, g, dinv, b, n)
    return h_out, dinv2


def _pool(h_out, dinv2, srcx, dst, src_flat, dst_flat, ew_flat, n, kk):
    n_pad = h_out.shape[0]
    k_pad = _pad_rows(kk)
    g2 = _g2(h_out, dinv2)
    B = _row_agg(n_pad, n_pad)(g2, srcx, dst,
                               jnp.zeros((n_pad, D), jnp.float32))
    sc = _score(h_out, B, dinv2, n)
    posx, mval = _select(sc, n, kk)
    Xh = _row_compact(n_pad, k_pad)(h_out, posx.reshape(n_pad),
                                    jnp.zeros((k_pad, D), jnp.float32))
    ro, X = _readout_combine(Xh, kk)
    mval_flat = mval.reshape(n_pad)
    ab = _elem_gather2(n_pad)(mval_flat, src_flat, dst_flat)
    s_n, d_n, ew_n, sx_n = _relabel(ab[:E_PAD].reshape(ER, 128),
                                    ab[E_PAD:].reshape(ER, 128),
                                    ew_flat.reshape(ER, 128), kk)
    return (ro, X, s_n.reshape(-1), d_n.reshape(-1), ew_n.reshape(-1),
            sx_n.reshape(-1))


def kernel(x, edge_index, batch, W1, b1, W2, b2, W3, b3,
           lin1_W, lin1_b, lin2_W, lin2_b, lin3_W, lin3_b):
    n = N_NODES
    n_pad = _pad_rows(n)
    pad_e = E_PAD - N_EDGES
    src = jnp.pad(edge_index[0].astype(jnp.int32), (0, pad_e),
                  constant_values=n)
    dst = jnp.pad(edge_index[1].astype(jnp.int32), (0, pad_e))
    ew = jnp.pad(jnp.ones((N_EDGES,), jnp.float32), (0, pad_e))
    x_pad = jnp.pad(x, ((0, n_pad - n), (0, 0)))

    h1, dinv21 = _layer(x_pad, W1, b1, src, dst, n)
    x1ro, X1, s1, d1, ew1, sx1 = _pool(
        h1, dinv21, src, dst, src, dst, ew, n, 5000)

    k1 = 5000
    h2, dinv22 = _layer(X1, W2, b2, sx1, d1, k1)
    x2ro, X2, s2, d2, ew2, sx2 = _pool(
        h2, dinv22, sx1, d1, s1, d1, ew1, k1, 2500)

    k2 = 2500
    h3, _ = _layer(X2, W3, b3, sx2, d2, k2)
    x3ro = _readout_single(h3, k2)

    return _head(x1ro, x2ro, x3ro, lin1_W, lin1_b, lin2_W, lin2_b,
                 lin3_W, lin3_b)


# speedup vs baseline: 35.4020x; 21.5338x over previous
import functools
import math

import jax
import jax.numpy as jnp
from jax import lax
from jax.experimental import pallas as pl
from jax.experimental.pallas import tpu as pltpu
from jax.experimental.pallas import tpu_sc as plsc

N_NODES = 10000
N_EDGES = 320000
D = 128
NW = 32
EC = 128
E_PAD = 327680
NBLK = E_PAD // EC
PER_W = NBLK // NW

_mesh = functools.partial(
    plsc.VectorSubcoreMesh, core_axis_name="c", subcore_axis_name="s")


def _pad_rows(n):
    return ((n + 255) // 256) * 256


@functools.lru_cache(None)
def _seg_count(n_pad):
    zr = n_pad // 16
    NB = 8
    G = PER_W // NB

    @functools.partial(
        pl.kernel, mesh=_mesh(),
        out_type=jax.ShapeDtypeStruct((2 * n_pad,), jnp.float32),
        scratch_types=(
            [pltpu.VMEM((EC,), jnp.int32) for _ in range(NB)]
            + [pltpu.VMEM((EC,), jnp.float32),
               pltpu.VMEM((n_pad // 16,), jnp.float32),
               pltpu.VMEM_SHARED((n_pad,), jnp.float32),
               pltpu.SemaphoreType.DMA,
               pltpu.SemaphoreType.DMA]
        ),
    )
    def k(idx_hbm, ones_hbm, zeros_hbm, out, *rest):
        idx_vs = rest[:NB]
        ones_v, buf_v, acc, sem_i, sem_s = rest[NB:]
        c = lax.axis_index("c")
        s = lax.axis_index("s")
        w = c * 16 + s
        pltpu.sync_copy(zeros_hbm.at[pl.ds(s * zr, zr)], buf_v)
        pltpu.sync_copy(buf_v, acc.at[pl.ds(s * zr, zr)])
        pltpu.sync_copy(ones_hbm, ones_v)
        plsc.subcore_barrier()

        def body(g, carry):
            j0 = (w * PER_W + g * NB) * EC
            loads = [pltpu.async_copy(
                idx_hbm.at[pl.ds(j0 + b * EC, EC)], idx_vs[b], sem_i)
                for b in range(NB)]
            for cp in loads:
                cp.wait()
            for b in range(NB):
                pltpu.sync_copy(ones_v, acc.at[idx_vs[b]], add=True)
            return carry

        lax.fori_loop(0, G, body, 0)
        plsc.subcore_barrier()
        pltpu.sync_copy(acc.at[pl.ds(s * zr, zr)], buf_v)
        pltpu.sync_copy(buf_v, out.at[pl.ds(c * n_pad + s * zr, zr)])

    return k


@functools.lru_cache(None)
def _row_agg(n_tab, n_out_pad):
    zr = n_out_pad // 16
    ECR = 64
    PER = E_PAD // (NW * ECR)
    NB = 4 if n_out_pad > 8192 else 10
    G = PER // NB

    @functools.partial(
        pl.kernel, mesh=_mesh(),
        out_type=jax.ShapeDtypeStruct((2, n_out_pad, D), jnp.float32),
        scratch_types=(
            [pltpu.VMEM((ECR,), jnp.int32) for _ in range(2 * NB)]
            + [pltpu.VMEM((ECR, D), jnp.float32) for _ in range(NB)]
            + [pltpu.VMEM_SHARED((n_out_pad, D), jnp.float32),
               pltpu.SemaphoreType.DMA,
               pltpu.SemaphoreType.DMA,
               pltpu.SemaphoreType.DMA]
        ),
    )
    def k(tab_hbm, src_hbm, dst_hbm, zeros_hbm, out, *rest):
        src_vs = rest[:NB]
        dst_vs = rest[NB:2 * NB]
        rows_vs = rest[2 * NB:3 * NB]
        acc, sem_i, sem_g, sem_s = rest[3 * NB:]
        c = lax.axis_index("c")
        s = lax.axis_index("s")
        w = c * 16 + s
        pltpu.sync_copy(zeros_hbm.at[pl.ds(s * zr, zr)], acc.at[pl.ds(s * zr, zr)])
        plsc.subcore_barrier()

        def body(g, carry):
            j0 = (w * PER + g * NB) * ECR
            loads = [pltpu.async_copy(
                src_hbm.at[pl.ds(j0 + b * ECR, ECR)], src_vs[b], sem_i)
                for b in range(NB)]
            loads += [pltpu.async_copy(
                dst_hbm.at[pl.ds(j0 + b * ECR, ECR)], dst_vs[b], sem_i)
                for b in range(NB)]
            for cp in loads:
                cp.wait()
            gathers = [pltpu.async_copy(tab_hbm.at[src_vs[b]], rows_vs[b],
                                        sem_g) for b in range(NB)]
            for cp in gathers:
                cp.wait()
            adds = [pltpu.async_copy(rows_vs[b], acc.at[dst_vs[b]], sem_s,
                                     add=True) for b in range(NB)]
            for cp in adds:
                cp.wait()
            return carry

        lax.fori_loop(0, G, body, 0)
        plsc.subcore_barrier()
        pltpu.sync_copy(acc.at[pl.ds(s * zr, zr)], out.at[c, pl.ds(s * zr, zr)])

    return k


@functools.lru_cache(None)
def _row_compact(n_pad, k_pad):
    T = 5
    ch = n_pad // (NW * T)
    zr = k_pad // 16

    @functools.partial(
        pl.kernel, mesh=_mesh(),
        out_type=jax.ShapeDtypeStruct((2, k_pad, D), jnp.float32),
        scratch_types=(
            [pltpu.VMEM((ch,), jnp.int32) for _ in range(T)]
            + [pltpu.VMEM((ch, D), jnp.float32) for _ in range(T)]
            + [pltpu.VMEM_SHARED((k_pad, D), jnp.float32),
               pltpu.SemaphoreType.DMA,
               pltpu.SemaphoreType.DMA]
        ),
    )
    def k(h_hbm, posx_hbm, zeros_hbm, out, *rest):
        posx_vs = rest[:T]
        rows_vs = rest[T:2 * T]
        acc, sem_i, sem_s = rest[2 * T:]
        c = lax.axis_index("c")
        s = lax.axis_index("s")
        w = c * 16 + s
        pltpu.sync_copy(zeros_hbm.at[pl.ds(s * zr, zr)], acc.at[pl.ds(s * zr, zr)])
        plsc.subcore_barrier()
        loads = [pltpu.async_copy(
            posx_hbm.at[pl.ds(w * (T * ch) + t * ch, ch)], posx_vs[t], sem_i)
            for t in range(T)]
        loads += [pltpu.async_copy(
            h_hbm.at[pl.ds(w * (T * ch) + t * ch, ch)], rows_vs[t], sem_i)
            for t in range(T)]
        for cp in loads:
            cp.wait()
        adds = [pltpu.async_copy(rows_vs[t], acc.at[posx_vs[t]], sem_s,
                                 add=True) for t in range(T)]
        for cp in adds:
            cp.wait()
        plsc.subcore_barrier()
        pltpu.sync_copy(acc.at[pl.ds(s * zr, zr)], out.at[c, pl.ds(s * zr, zr)])

    return k


@functools.lru_cache(None)
def _elem_gather2(n_tab):
    NB = 8
    zr = n_tab // 16

    @functools.partial(
        pl.kernel, mesh=_mesh(),
        out_type=jax.ShapeDtypeStruct((2 * E_PAD,), jnp.int32),
        scratch_types=(
            [pltpu.VMEM((EC,), jnp.int32) for _ in range(2 * NB)]
            + [pltpu.VMEM((zr,), jnp.int32),
               pltpu.VMEM_SHARED((n_tab,), jnp.int32),
               pltpu.SemaphoreType.DMA,
               pltpu.SemaphoreType.DMA,
               pltpu.SemaphoreType.DMA]
        ),
    )
    def k(tab_hbm, src_hbm, dst_hbm, out, *rest):
        idx_vs = rest[:NB]
        g_vs = rest[NB:2 * NB]
        tbuf, tab_s, sem_i, sem_g, sem_o = rest[2 * NB:]
        c = lax.axis_index("c")
        s = lax.axis_index("s")
        w = c * 16 + s
        pltpu.sync_copy(tab_hbm.at[pl.ds(s * zr, zr)], tbuf)
        pltpu.sync_copy(tbuf, tab_s.at[pl.ds(s * zr, zr)])
        plsc.subcore_barrier()

        def make_body(idx_ref, half):
            def body(g, carry):
                base0 = (w * PER_W + g * NB) * EC
                loads = [pltpu.async_copy(
                    idx_ref.at[pl.ds(base0 + b * EC, EC)], idx_vs[b], sem_i)
                    for b in range(NB)]
                for cp in loads:
                    cp.wait()
                gathers = [pltpu.async_copy(tab_s.at[idx_vs[b]], g_vs[b],
                                            sem_g) for b in range(NB)]
                for cp in gathers:
                    cp.wait()
                stores = [pltpu.async_copy(
                    g_vs[b], out.at[pl.ds(half * E_PAD + base0 + b * EC, EC)],
                    sem_o) for b in range(NB)]
                for cp in stores:
                    cp.wait()
                return carry
            return body

        lax.fori_loop(0, PER_W // NB, make_body(src_hbm, 0), 0)
        lax.fori_loop(0, PER_W // NB, make_body(dst_hbm, 1), 0)

    return k


def _tc(body, out_shape):
    return pl.pallas_call(body, out_shape=out_shape)


def _mm(x, w):
    def body(x_ref, w_ref, o_ref):
        o_ref[...] = jnp.dot(x_ref[...], w_ref[...],
                             preferred_element_type=jnp.float32)
    return _tc(body, jax.ShapeDtypeStruct((x.shape[0], w.shape[1]),
                                          jnp.float32))(x, w)


def _pre(deg2, h):
    n_pad = h.shape[0]

    def body(d_ref, h_ref, g_ref, di_ref, di2_ref):
        d2 = d_ref[0, :] + d_ref[1, :]
        dinv = lax.rsqrt(d2 + 1.0)
        dinv2 = jnp.where(d2 > 0, lax.rsqrt(jnp.where(d2 > 0, d2, 1.0)), 0.0)
        di_ref[...] = dinv
        di2_ref[...] = dinv2
        g_ref[...] = dinv[:, None] * h_ref[...]

    return _tc(body, (jax.ShapeDtypeStruct((n_pad, D), jnp.float32),
                      jax.ShapeDtypeStruct((n_pad,), jnp.float32),
                      jax.ShapeDtypeStruct((n_pad,), jnp.float32)))(deg2, h)


def _post_gcn(A, g, dinv, b, n):
    n_pad = g.shape[0]

    def body(a_ref, g_ref, di_ref, b_ref, o_ref):
        acc = a_ref[0] + a_ref[1] + g_ref[...]
        h = jax.nn.relu(di_ref[...][:, None] * acc + b_ref[...][None, :])
        row = lax.broadcasted_iota(jnp.int32, (n_pad, D), 0)
        o_ref[...] = jnp.where(row < n, h, 0.0)

    return _tc(body, jax.ShapeDtypeStruct((n_pad, D), jnp.float32))(
        A, g, dinv, b)


def _g2(h, dinv2):
    n_pad = h.shape[0]

    def body(h_ref, di2_ref, o_ref):
        o_ref[...] = di2_ref[...][:, None] * h_ref[...]

    return _tc(body, jax.ShapeDtypeStruct((n_pad, D), jnp.float32))(h, dinv2)


def _score(h, B, dinv2, n):
    n_pad = h.shape[0]
    R = n_pad // 128

    def body(h_ref, b_ref, di2_ref, o_ref):
        agg = h_ref[...] - di2_ref[...][:, None] * (b_ref[0] + b_ref[1])
        sc = jnp.sum(jnp.abs(agg), axis=1).reshape(R, 128)
        idx = (lax.broadcasted_iota(jnp.int32, (R, 128), 0) * 128
               + lax.broadcasted_iota(jnp.int32, (R, 128), 1))
        o_ref[...] = jnp.where(idx < n, sc, -jnp.inf)

    return _tc(body, jax.ShapeDtypeStruct((R, 128), jnp.float32))(h, B, dinv2)


def _prefix_excl(m):
    R = m.shape[0]
    jj = lax.broadcasted_iota(jnp.int32, (128, 128), 0)
    kk = lax.broadcasted_iota(jnp.int32, (128, 128), 1)
    lincl = (jj <= kk).astype(jnp.float32)
    lane_incl = jnp.dot(m, lincl, preferred_element_type=jnp.float32)
    lane_excl = lane_incl - m
    totals = jnp.sum(m, axis=1)
    totv = jnp.concatenate([totals, jnp.zeros((128 - R,), jnp.float32)])
    strict = (kk < jj).astype(jnp.float32)
    offs = jnp.sum(strict * totv[None, :], axis=1)[:R]
    return offs[:, None] + lane_excl


def _select(score, n, k):
    R = score.shape[0]
    k_pad = _pad_rows(k)

    def body(s_ref, posx_ref, mval_ref):
        s = s_ref[...]
        idx = (lax.broadcasted_iota(jnp.int32, (R, 128), 0) * 128
               + lax.broadcasted_iota(jnp.int32, (R, 128), 1))
        mask = idx < n
        key = lax.bitcast_convert_type(s, jnp.int32)
        key = jnp.where(mask, key, -1)

        def step(_, lohi):
            lo, hi = lohi
            d = hi - lo
            mid = lo + d // 2 + (d & 1)
            cnt = jnp.sum((key >= mid).astype(jnp.float32))
            big = cnt >= k
            return (jnp.where(big, mid, lo), jnp.where(big, hi, mid - 1))

        lo, hi = lax.fori_loop(0, 31, step, (jnp.int32(0), jnp.int32(2**31 - 1)))
        tau = lo
        gt = key > tau
        cnt_gt = jnp.sum(gt.astype(jnp.float32))
        tie = key == tau
        tie_rank = _prefix_excl(tie.astype(jnp.float32))
        selected = gt | (tie & (tie_rank < (k - cnt_gt)))
        mapping = _prefix_excl(selected.astype(jnp.float32)).astype(jnp.int32)
        dummy = k + idx % (k_pad - k)
        posx_ref[...] = jnp.where(selected, mapping, dummy)
        mval_ref[...] = jnp.where(selected, mapping, -1)

    return _tc(body, (jax.ShapeDtypeStruct((R, 128), jnp.int32),
                      jax.ShapeDtypeStruct((R, 128), jnp.int32)))(score)


def _relabel(a, b, ew, kk):
    E2 = a.shape
    k_pad = _pad_rows(kk)
    ps = k_pad - kk

    def body(a_ref, b_ref, w_ref, s_ref, d_ref, wo_ref, sx_ref, dx_ref):
        av, bv, wv = a_ref[...], b_ref[...], w_ref[...]
        valid = (av >= 0) & (bv >= 0) & (wv > 0)
        e = (lax.broadcasted_iota(jnp.int32, E2, 0) * 128
             + lax.broadcasted_iota(jnp.int32, E2, 1))
        dummy = kk + e % ps
        s_ref[...] = jnp.where(valid, av, 0)
        d_ref[...] = jnp.where(valid, bv, 0)
        wo_ref[...] = valid.astype(jnp.float32)
        sx_ref[...] = jnp.where(valid, av, dummy)
        dx_ref[...] = jnp.where(valid, bv, dummy)

    return _tc(body, (jax.ShapeDtypeStruct(E2, jnp.int32),
                      jax.ShapeDtypeStruct(E2, jnp.int32),
                      jax.ShapeDtypeStruct(E2, jnp.float32),
                      jax.ShapeDtypeStruct(E2, jnp.int32),
                      jax.ShapeDtypeStruct(E2, jnp.int32)))(a, b, ew)


def _readout_combine(Xh, kk):
    n_pad = Xh.shape[1]

    def body(x_ref, ro_ref, xo_ref):
        x = x_ref[0] + x_ref[1]
        row = lax.broadcasted_iota(jnp.int32, (n_pad, D), 0)
        xm = jnp.where(row < kk, x, 0.0)
        xo_ref[...] = xm
        mx = jnp.max(jnp.where(row < kk, x, -jnp.inf), axis=0)
        mn = jnp.sum(xm, axis=0) / kk
        ro_ref[...] = jnp.concatenate([mx, mn]).reshape(1, 2 * D)

    return _tc(body, (jax.ShapeDtypeStruct((1, 2 * D), jnp.float32),
                      jax.ShapeDtypeStruct((n_pad, D), jnp.float32)))(Xh)


def _readout_single(X, kk):
    n_pad = X.shape[0]

    def body(x_ref, ro_ref):
        x = x_ref[...]
        row = lax.broadcasted_iota(jnp.int32, (n_pad, D), 0)
        mx = jnp.max(jnp.where(row < kk, x, -jnp.inf), axis=0)
        mn = jnp.sum(jnp.where(row < kk, x, 0.0), axis=0) / kk
        ro_ref[...] = jnp.concatenate([mx, mn]).reshape(1, 2 * D)

    return _tc(body, jax.ShapeDtypeStruct((1, 2 * D), jnp.float32))(X)


def _head(x1, x2, x3, lin1_W, lin1_b, lin2_W, lin2_b, lin3_W, lin3_b):
    def body(x1_ref, x2_ref, x3_ref, w1_ref, b1_ref, w2_ref, b2_ref,
             w3_ref, b3_ref, o_ref):
        z = (jax.nn.relu(x1_ref[...]) + jax.nn.relu(x2_ref[...])
             + jax.nn.relu(x3_ref[...]))
        z = jax.nn.relu(jnp.dot(z, w1_ref[...],
                                preferred_element_type=jnp.float32)
                        + b1_ref[...][None, :])
        z = jax.nn.relu(jnp.dot(z, w2_ref[...],
                                preferred_element_type=jnp.float32)
                        + b2_ref[...][None, :])
        z = (jnp.dot(z, w3_ref[...], preferred_element_type=jnp.float32)
             + b3_ref[...][None, :])
        m = jnp.max(z, axis=-1, keepdims=True)
        e = jnp.exp(z - m)
        o_ref[...] = z - m - jnp.log(jnp.sum(e, axis=-1, keepdims=True))

    return _tc(body, jax.ShapeDtypeStruct((1, 6), jnp.float32))(
        x1, x2, x3, lin1_W, lin1_b, lin2_W, lin2_b, lin3_W, lin3_b)


ER = E_PAD // 128


def _layer(x_pad, W, b, srcx, dst, n):
    n_pad = x_pad.shape[0]
    zeros1 = jnp.zeros((n_pad,), jnp.float32)
    zeros2 = jnp.zeros((n_pad, D), jnp.float32)
    ones_ec = jnp.ones((EC,), jnp.float32)
    h = _mm(x_pad, W)
    deg = _seg_count(n_pad)(srcx, ones_ec, zeros1).reshape(2, n_pad)
    g, dinv, dinv2 = _pre(deg, h)
    A = _row_agg(n_pad, n_pad)(g, srcx, dst, zeros2)
    h_out = _post_gcn(A, g, dinv, b, n)
    return h_out, dinv2


def _pool(h_out, dinv2, srcx, dstx, src_flat, dst_flat, ew_flat, n, kk):
    n_pad = h_out.shape[0]
    k_pad = _pad_rows(kk)
    g2 = _g2(h_out, dinv2)
    B = _row_agg(n_pad, n_pad)(g2, srcx, dstx,
                               jnp.zeros((n_pad, D), jnp.float32))
    sc = _score(h_out, B, dinv2, n)
    posx, mval = _select(sc, n, kk)
    Xh = _row_compact(n_pad, k_pad)(h_out, posx.reshape(n_pad),
                                    jnp.zeros((k_pad, D), jnp.float32))
    ro, X = _readout_combine(Xh, kk)
    mval_flat = mval.reshape(n_pad)
    ab = _elem_gather2(n_pad)(mval_flat, src_flat, dst_flat)
    s_n, d_n, ew_n, sx_n, dx_n = _relabel(ab[:E_PAD].reshape(ER, 128),
                                          ab[E_PAD:].reshape(ER, 128),
                                          ew_flat.reshape(ER, 128), kk)
    return (ro, X, s_n.reshape(-1), d_n.reshape(-1), ew_n.reshape(-1),
            sx_n.reshape(-1), dx_n.reshape(-1))


def kernel(x, edge_index, batch, W1, b1, W2, b2, W3, b3,
           lin1_W, lin1_b, lin2_W, lin2_b, lin3_W, lin3_b):
    n = N_NODES
    n_pad = _pad_rows(n)
    pad_e = E_PAD - N_EDGES
    spread = n + jnp.arange(pad_e, dtype=jnp.int32) % (n_pad - n)
    src = jnp.concatenate([edge_index[0].astype(jnp.int32), spread])
    dst = jnp.concatenate([edge_index[1].astype(jnp.int32), spread])
    ew = jnp.pad(jnp.ones((N_EDGES,), jnp.float32), (0, pad_e))
    x_pad = jnp.pad(x, ((0, n_pad - n), (0, 0)))

    h1, dinv21 = _layer(x_pad, W1, b1, src, dst, n)
    x1ro, X1, s1, d1, ew1, sx1, dx1 = _pool(
        h1, dinv21, src, dst, src, dst, ew, n, 5000)

    k1 = 5000
    h2, dinv22 = _layer(X1, W2, b2, sx1, dx1, k1)
    x2ro, X2, s2, d2, ew2, sx2, dx2 = _pool(
        h2, dinv22, sx1, dx1, s1, d1, ew1, k1, 2500)

    k2 = 2500
    h3, _ = _layer(X2, W3, b3, sx2, dx2, k2)
    x3ro = _readout_single(h3, k2)

    return _head(x1ro, x2ro, x3ro, lin1_W, lin1_b, lin2_W, lin2_b,
                 lin3_W, lin3_b)


# speedup vs baseline: 35.7303x; 1.0093x over previous
import functools
import math

import jax
import jax.numpy as jnp
from jax import lax
from jax.experimental import pallas as pl
from jax.experimental.pallas import tpu as pltpu
from jax.experimental.pallas import tpu_sc as plsc

N_NODES = 10000
N_EDGES = 320000
D = 128
NW = 32
EC = 128
E_PAD = 327680
NBLK = E_PAD // EC
PER_W = NBLK // NW

_mesh = functools.partial(
    plsc.VectorSubcoreMesh, core_axis_name="c", subcore_axis_name="s")


def _pad_rows(n):
    return ((n + 255) // 256) * 256


@functools.lru_cache(None)
def _seg_count(n_pad):
    zr = n_pad // 16
    NB = 16
    G = PER_W // NB

    @functools.partial(
        pl.kernel, mesh=_mesh(),
        out_type=jax.ShapeDtypeStruct((2 * n_pad,), jnp.float32),
        scratch_types=(
            [pltpu.VMEM((EC,), jnp.int32) for _ in range(NB)]
            + [pltpu.VMEM((EC,), jnp.float32),
               pltpu.VMEM((n_pad // 16,), jnp.float32),
               pltpu.VMEM_SHARED((n_pad,), jnp.float32),
               pltpu.SemaphoreType.DMA,
               pltpu.SemaphoreType.DMA]
        ),
    )
    def k(idx_hbm, ones_hbm, zeros_hbm, out, *rest):
        idx_vs = rest[:NB]
        ones_v, buf_v, acc, sem_i, sem_s = rest[NB:]
        c = lax.axis_index("c")
        s = lax.axis_index("s")
        w = c * 16 + s
        pltpu.sync_copy(zeros_hbm.at[pl.ds(s * zr, zr)], buf_v)
        pltpu.sync_copy(buf_v, acc.at[pl.ds(s * zr, zr)])
        pltpu.sync_copy(ones_hbm, ones_v)
        plsc.subcore_barrier()

        def body(g, carry):
            j0 = (w * PER_W + g * NB) * EC
            loads = [pltpu.async_copy(
                idx_hbm.at[pl.ds(j0 + b * EC, EC)], idx_vs[b], sem_i)
                for b in range(NB)]
            for cp in loads:
                cp.wait()
            for b in range(NB):
                pltpu.sync_copy(ones_v, acc.at[idx_vs[b]], add=True)
            return carry

        lax.fori_loop(0, G, body, 0)
        plsc.subcore_barrier()
        pltpu.sync_copy(acc.at[pl.ds(s * zr, zr)], buf_v)
        pltpu.sync_copy(buf_v, out.at[pl.ds(c * n_pad + s * zr, zr)])

    return k


@functools.lru_cache(None)
def _row_agg(n_tab, n_out_pad):
    zr = n_out_pad // 16
    ECR = 64
    PER = E_PAD // (NW * ECR)
    NB = 4 if n_out_pad > 8192 else 10
    G = PER // NB

    @functools.partial(
        pl.kernel, mesh=_mesh(),
        out_type=jax.ShapeDtypeStruct((2, n_out_pad, D), jnp.float32),
        scratch_types=(
            [pltpu.VMEM((ECR,), jnp.int32) for _ in range(2 * NB)]
            + [pltpu.VMEM((ECR, D), jnp.float32) for _ in range(NB)]
            + [pltpu.VMEM_SHARED((n_out_pad, D), jnp.float32),
               pltpu.SemaphoreType.DMA,
               pltpu.SemaphoreType.DMA,
               pltpu.SemaphoreType.DMA]
        ),
    )
    def k(tab_hbm, src_hbm, dst_hbm, zeros_hbm, out, *rest):
        src_vs = rest[:NB]
        dst_vs = rest[NB:2 * NB]
        rows_vs = rest[2 * NB:3 * NB]
        acc, sem_i, sem_g, sem_s = rest[3 * NB:]
        c = lax.axis_index("c")
        s = lax.axis_index("s")
        w = c * 16 + s
        pltpu.sync_copy(zeros_hbm.at[pl.ds(s * zr, zr)], acc.at[pl.ds(s * zr, zr)])
        plsc.subcore_barrier()

        def body(g, carry):
            j0 = (w * PER + g * NB) * ECR
            loads = [pltpu.async_copy(
                src_hbm.at[pl.ds(j0 + b * ECR, ECR)], src_vs[b], sem_i)
                for b in range(NB)]
            loads += [pltpu.async_copy(
                dst_hbm.at[pl.ds(j0 + b * ECR, ECR)], dst_vs[b], sem_i)
                for b in range(NB)]
            for cp in loads:
                cp.wait()
            gathers = [pltpu.async_copy(tab_hbm.at[src_vs[b]], rows_vs[b],
                                        sem_g) for b in range(NB)]
            for cp in gathers:
                cp.wait()
            adds = [pltpu.async_copy(rows_vs[b], acc.at[dst_vs[b]], sem_s,
                                     add=True) for b in range(NB)]
            for cp in adds:
                cp.wait()
            return carry

        lax.fori_loop(0, G, body, 0)
        plsc.subcore_barrier()
        pltpu.sync_copy(acc.at[pl.ds(s * zr, zr)], out.at[c, pl.ds(s * zr, zr)])

    return k


@functools.lru_cache(None)
def _row_compact(n_pad, k_pad):
    T = 5
    ch = n_pad // (NW * T)
    zr = k_pad // 16

    @functools.partial(
        pl.kernel, mesh=_mesh(),
        out_type=jax.ShapeDtypeStruct((2, k_pad, D), jnp.float32),
        scratch_types=(
            [pltpu.VMEM((ch,), jnp.int32) for _ in range(T)]
            + [pltpu.VMEM((ch, D), jnp.float32) for _ in range(T)]
            + [pltpu.VMEM_SHARED((k_pad, D), jnp.float32),
               pltpu.SemaphoreType.DMA,
               pltpu.SemaphoreType.DMA]
        ),
    )
    def k(h_hbm, posx_hbm, zeros_hbm, out, *rest):
        posx_vs = rest[:T]
        rows_vs = rest[T:2 * T]
        acc, sem_i, sem_s = rest[2 * T:]
        c = lax.axis_index("c")
        s = lax.axis_index("s")
        w = c * 16 + s
        pltpu.sync_copy(zeros_hbm.at[pl.ds(s * zr, zr)], acc.at[pl.ds(s * zr, zr)])
        plsc.subcore_barrier()
        loads = [pltpu.async_copy(
            posx_hbm.at[pl.ds(w * (T * ch) + t * ch, ch)], posx_vs[t], sem_i)
            for t in range(T)]
        loads += [pltpu.async_copy(
            h_hbm.at[pl.ds(w * (T * ch) + t * ch, ch)], rows_vs[t], sem_i)
            for t in range(T)]
        for cp in loads:
            cp.wait()
        adds = [pltpu.async_copy(rows_vs[t], acc.at[posx_vs[t]], sem_s,
                                 add=True) for t in range(T)]
        for cp in adds:
            cp.wait()
        plsc.subcore_barrier()
        pltpu.sync_copy(acc.at[pl.ds(s * zr, zr)], out.at[c, pl.ds(s * zr, zr)])

    return k


@functools.lru_cache(None)
def _elem_gather2(n_tab):
    NB = 16
    zr = n_tab // 16

    @functools.partial(
        pl.kernel, mesh=_mesh(),
        out_type=jax.ShapeDtypeStruct((2 * E_PAD,), jnp.int32),
        scratch_types=(
            [pltpu.VMEM((EC,), jnp.int32) for _ in range(2 * NB)]
            + [pltpu.VMEM((zr,), jnp.int32),
               pltpu.VMEM_SHARED((n_tab,), jnp.int32),
               pltpu.SemaphoreType.DMA,
               pltpu.SemaphoreType.DMA,
               pltpu.SemaphoreType.DMA]
        ),
    )
    def k(tab_hbm, src_hbm, dst_hbm, out, *rest):
        idx_vs = rest[:NB]
        g_vs = rest[NB:2 * NB]
        tbuf, tab_s, sem_i, sem_g, sem_o = rest[2 * NB:]
        c = lax.axis_index("c")
        s = lax.axis_index("s")
        w = c * 16 + s
        pltpu.sync_copy(tab_hbm.at[pl.ds(s * zr, zr)], tbuf)
        pltpu.sync_copy(tbuf, tab_s.at[pl.ds(s * zr, zr)])
        plsc.subcore_barrier()

        def make_body(idx_ref, half):
            def body(g, carry):
                base0 = (w * PER_W + g * NB) * EC
                loads = [pltpu.async_copy(
                    idx_ref.at[pl.ds(base0 + b * EC, EC)], idx_vs[b], sem_i)
                    for b in range(NB)]
                for cp in loads:
                    cp.wait()
                gathers = [pltpu.async_copy(tab_s.at[idx_vs[b]], g_vs[b],
                                            sem_g) for b in range(NB)]
                for cp in gathers:
                    cp.wait()
                stores = [pltpu.async_copy(
                    g_vs[b], out.at[pl.ds(half * E_PAD + base0 + b * EC, EC)],
                    sem_o) for b in range(NB)]
                for cp in stores:
                    cp.wait()
                return carry
            return body

        lax.fori_loop(0, PER_W // NB, make_body(src_hbm, 0), 0)
        lax.fori_loop(0, PER_W // NB, make_body(dst_hbm, 1), 0)

    return k


def _tc(body, out_shape):
    return pl.pallas_call(body, out_shape=out_shape)


def _mm(x, w):
    def body(x_ref, w_ref, o_ref):
        o_ref[...] = jnp.dot(x_ref[...], w_ref[...],
                             preferred_element_type=jnp.float32)
    return _tc(body, jax.ShapeDtypeStruct((x.shape[0], w.shape[1]),
                                          jnp.float32))(x, w)


def _pre(deg2, h):
    n_pad = h.shape[0]

    def body(d_ref, h_ref, g_ref, di_ref, di2_ref):
        d2 = d_ref[0, :] + d_ref[1, :]
        dinv = lax.rsqrt(d2 + 1.0)
        dinv2 = jnp.where(d2 > 0, lax.rsqrt(jnp.where(d2 > 0, d2, 1.0)), 0.0)
        di_ref[...] = dinv
        di2_ref[...] = dinv2
        g_ref[...] = dinv[:, None] * h_ref[...]

    return _tc(body, (jax.ShapeDtypeStruct((n_pad, D), jnp.float32),
                      jax.ShapeDtypeStruct((n_pad,), jnp.float32),
                      jax.ShapeDtypeStruct((n_pad,), jnp.float32)))(deg2, h)


def _post_gcn(A, g, dinv, b, n):
    n_pad = g.shape[0]

    def body(a_ref, g_ref, di_ref, b_ref, o_ref):
        acc = a_ref[0] + a_ref[1] + g_ref[...]
        h = jax.nn.relu(di_ref[...][:, None] * acc + b_ref[...][None, :])
        row = lax.broadcasted_iota(jnp.int32, (n_pad, D), 0)
        o_ref[...] = jnp.where(row < n, h, 0.0)

    return _tc(body, jax.ShapeDtypeStruct((n_pad, D), jnp.float32))(
        A, g, dinv, b)


def _g2(h, dinv2):
    n_pad = h.shape[0]

    def body(h_ref, di2_ref, o_ref):
        o_ref[...] = di2_ref[...][:, None] * h_ref[...]

    return _tc(body, jax.ShapeDtypeStruct((n_pad, D), jnp.float32))(h, dinv2)


def _score(h, B, dinv2, n):
    n_pad = h.shape[0]
    R = n_pad // 128

    def body(h_ref, b_ref, di2_ref, o_ref):
        agg = h_ref[...] - di2_ref[...][:, None] * (b_ref[0] + b_ref[1])
        sc = jnp.sum(jnp.abs(agg), axis=1).reshape(R, 128)
        idx = (lax.broadcasted_iota(jnp.int32, (R, 128), 0) * 128
               + lax.broadcasted_iota(jnp.int32, (R, 128), 1))
        o_ref[...] = jnp.where(idx < n, sc, -jnp.inf)

    return _tc(body, jax.ShapeDtypeStruct((R, 128), jnp.float32))(h, B, dinv2)


def _prefix_excl(m):
    R = m.shape[0]
    jj = lax.broadcasted_iota(jnp.int32, (128, 128), 0)
    kk = lax.broadcasted_iota(jnp.int32, (128, 128), 1)
    lincl = (jj <= kk).astype(jnp.float32)
    lane_incl = jnp.dot(m, lincl, preferred_element_type=jnp.float32)
    lane_excl = lane_incl - m
    totals = jnp.sum(m, axis=1)
    totv = jnp.concatenate([totals, jnp.zeros((128 - R,), jnp.float32)])
    strict = (kk < jj).astype(jnp.float32)
    offs = jnp.sum(strict * totv[None, :], axis=1)[:R]
    return offs[:, None] + lane_excl


def _select(score, n, k):
    R = score.shape[0]
    k_pad = _pad_rows(k)

    def body(s_ref, posx_ref, mval_ref):
        s = s_ref[...]
        idx = (lax.broadcasted_iota(jnp.int32, (R, 128), 0) * 128
               + lax.broadcasted_iota(jnp.int32, (R, 128), 1))
        mask = idx < n
        key = lax.bitcast_convert_type(s, jnp.int32)
        key = jnp.where(mask, key, -1)

        def step(_, lohi):
            lo, hi = lohi
            d = hi - lo
            mid = lo + d // 2 + (d & 1)
            cnt = jnp.sum((key >= mid).astype(jnp.float32))
            big = cnt >= k
            return (jnp.where(big, mid, lo), jnp.where(big, hi, mid - 1))

        lo, hi = lax.fori_loop(0, 31, step, (jnp.int32(0), jnp.int32(2**31 - 1)))
        tau = lo
        gt = key > tau
        cnt_gt = jnp.sum(gt.astype(jnp.float32))
        tie = key == tau
        tie_rank = _prefix_excl(tie.astype(jnp.float32))
        selected = gt | (tie & (tie_rank < (k - cnt_gt)))
        mapping = _prefix_excl(selected.astype(jnp.float32)).astype(jnp.int32)
        dummy = k + idx % (k_pad - k)
        posx_ref[...] = jnp.where(selected, mapping, dummy)
        mval_ref[...] = jnp.where(selected, mapping, -1)

    return _tc(body, (jax.ShapeDtypeStruct((R, 128), jnp.int32),
                      jax.ShapeDtypeStruct((R, 128), jnp.int32)))(score)


def _relabel(a, b, ew, kk):
    E2 = a.shape
    k_pad = _pad_rows(kk)
    ps = k_pad - kk

    def body(a_ref, b_ref, w_ref, s_ref, d_ref, wo_ref, sx_ref, dx_ref):
        av, bv, wv = a_ref[...], b_ref[...], w_ref[...]
        valid = (av >= 0) & (bv >= 0) & (wv > 0)
        e = (lax.broadcasted_iota(jnp.int32, E2, 0) * 128
             + lax.broadcasted_iota(jnp.int32, E2, 1))
        dummy = kk + e % ps
        s_ref[...] = jnp.where(valid, av, 0)
        d_ref[...] = jnp.where(valid, bv, 0)
        wo_ref[...] = valid.astype(jnp.float32)
        sx_ref[...] = jnp.where(valid, av, dummy)
        dx_ref[...] = jnp.where(valid, bv, dummy)

    return _tc(body, (jax.ShapeDtypeStruct(E2, jnp.int32),
                      jax.ShapeDtypeStruct(E2, jnp.int32),
                      jax.ShapeDtypeStruct(E2, jnp.float32),
                      jax.ShapeDtypeStruct(E2, jnp.int32),
                      jax.ShapeDtypeStruct(E2, jnp.int32)))(a, b, ew)


def _readout_combine(Xh, kk):
    n_pad = Xh.shape[1]

    def body(x_ref, ro_ref, xo_ref):
        x = x_ref[0] + x_ref[1]
        row = lax.broadcasted_iota(jnp.int32, (n_pad, D), 0)
        xm = jnp.where(row < kk, x, 0.0)
        xo_ref[...] = xm
        mx = jnp.max(jnp.where(row < kk, x, -jnp.inf), axis=0)
        mn = jnp.sum(xm, axis=0) / kk
        ro_ref[...] = jnp.concatenate([mx, mn]).reshape(1, 2 * D)

    return _tc(body, (jax.ShapeDtypeStruct((1, 2 * D), jnp.float32),
                      jax.ShapeDtypeStruct((n_pad, D), jnp.float32)))(Xh)


def _readout_single(X, kk):
    n_pad = X.shape[0]

    def body(x_ref, ro_ref):
        x = x_ref[...]
        row = lax.broadcasted_iota(jnp.int32, (n_pad, D), 0)
        mx = jnp.max(jnp.where(row < kk, x, -jnp.inf), axis=0)
        mn = jnp.sum(jnp.where(row < kk, x, 0.0), axis=0) / kk
        ro_ref[...] = jnp.concatenate([mx, mn]).reshape(1, 2 * D)

    return _tc(body, jax.ShapeDtypeStruct((1, 2 * D), jnp.float32))(X)


def _head(x1, x2, x3, lin1_W, lin1_b, lin2_W, lin2_b, lin3_W, lin3_b):
    def body(x1_ref, x2_ref, x3_ref, w1_ref, b1_ref, w2_ref, b2_ref,
             w3_ref, b3_ref, o_ref):
        z = (jax.nn.relu(x1_ref[...]) + jax.nn.relu(x2_ref[...])
             + jax.nn.relu(x3_ref[...]))
        z = jax.nn.relu(jnp.dot(z, w1_ref[...],
                                preferred_element_type=jnp.float32)
                        + b1_ref[...][None, :])
        z = jax.nn.relu(jnp.dot(z, w2_ref[...],
                                preferred_element_type=jnp.float32)
                        + b2_ref[...][None, :])
        z = (jnp.dot(z, w3_ref[...], preferred_element_type=jnp.float32)
             + b3_ref[...][None, :])
        m = jnp.max(z, axis=-1, keepdims=True)
        e = jnp.exp(z - m)
        o_ref[...] = z - m - jnp.log(jnp.sum(e, axis=-1, keepdims=True))

    return _tc(body, jax.ShapeDtypeStruct((1, 6), jnp.float32))(
        x1, x2, x3, lin1_W, lin1_b, lin2_W, lin2_b, lin3_W, lin3_b)


ER = E_PAD // 128


def _layer(x_pad, W, b, srcx, dst, n):
    n_pad = x_pad.shape[0]
    zeros1 = jnp.zeros((n_pad,), jnp.float32)
    zeros2 = jnp.zeros((n_pad, D), jnp.float32)
    ones_ec = jnp.ones((EC,), jnp.float32)
    h = _mm(x_pad, W)
    deg = _seg_count(n_pad)(srcx, ones_ec, zeros1).reshape(2, n_pad)
    g, dinv, dinv2 = _pre(deg, h)
    A = _row_agg(n_pad, n_pad)(g, srcx, dst, zeros2)
    h_out = _post_gcn(A, g, dinv, b, n)
    return h_out, dinv2


def _pool(h_out, dinv2, srcx, dstx, src_flat, dst_flat, ew_flat, n, kk):
    n_pad = h_out.shape[0]
    k_pad = _pad_rows(kk)
    g2 = _g2(h_out, dinv2)
    B = _row_agg(n_pad, n_pad)(g2, srcx, dstx,
                               jnp.zeros((n_pad, D), jnp.float32))
    sc = _score(h_out, B, dinv2, n)
    posx, mval = _select(sc, n, kk)
    Xh = _row_compact(n_pad, k_pad)(h_out, posx.reshape(n_pad),
                                    jnp.zeros((k_pad, D), jnp.float32))
    ro, X = _readout_combine(Xh, kk)
    mval_flat = mval.reshape(n_pad)
    ab = _elem_gather2(n_pad)(mval_flat, src_flat, dst_flat)
    s_n, d_n, ew_n, sx_n, dx_n = _relabel(ab[:E_PAD].reshape(ER, 128),
                                          ab[E_PAD:].reshape(ER, 128),
                                          ew_flat.reshape(ER, 128), kk)
    return (ro, X, s_n.reshape(-1), d_n.reshape(-1), ew_n.reshape(-1),
            sx_n.reshape(-1), dx_n.reshape(-1))


def kernel(x, edge_index, batch, W1, b1, W2, b2, W3, b3,
           lin1_W, lin1_b, lin2_W, lin2_b, lin3_W, lin3_b):
    n = N_NODES
    n_pad = _pad_rows(n)
    pad_e = E_PAD - N_EDGES
    spread = n + jnp.arange(pad_e, dtype=jnp.int32) % (n_pad - n)
    src = jnp.concatenate([edge_index[0].astype(jnp.int32), spread])
    dst = jnp.concatenate([edge_index[1].astype(jnp.int32), spread])
    ew = jnp.pad(jnp.ones((N_EDGES,), jnp.float32), (0, pad_e))
    x_pad = jnp.pad(x, ((0, n_pad - n), (0, 0)))

    h1, dinv21 = _layer(x_pad, W1, b1, src, dst, n)
    x1ro, X1, s1, d1, ew1, sx1, dx1 = _pool(
        h1, dinv21, src, dst, src, dst, ew, n, 5000)

    k1 = 5000
    h2, dinv22 = _layer(X1, W2, b2, sx1, dx1, k1)
    x2ro, X2, s2, d2, ew2, sx2, dx2 = _pool(
        h2, dinv22, sx1, dx1, s1, d1, ew1, k1, 2500)

    k2 = 2500
    h3, _ = _layer(X2, W3, b3, sx2, dx2, k2)
    x3ro = _readout_single(h3, k2)

    return _head(x1ro, x2ro, x3ro, lin1_W, lin1_b, lin2_W, lin2_b,
                 lin3_W, lin3_b)
